# Initial kernel scaffold; baseline (speedup 1.0000x reference)
import jax
import jax.numpy as jnp
from jax.experimental import pallas as pl


def kernel(x, gate_w, w1, w2):
    raise NotImplementedError("write your pallas kernel here")


# speedup vs baseline: 1.4005x; 1.4005x over previous
import functools
import math

import jax
import jax.numpy as jnp
from jax import lax
from jax.experimental import pallas as pl

NUM_EXPERTS = 16
N_EMBD = 768
N_HIDDEN = 1024


def _gate_body(x_ref, gw_ref, probs_ref):
    xb = x_ref[0]
    gw = gw_ref[...]
    s = lax.dot_general(xb, gw, (((1,), (1,)), ((), ())),
                        preferred_element_type=jnp.float32)
    m = jnp.max(s, axis=1, keepdims=True)
    e = jnp.exp(s - m)
    p = e / jnp.sum(e, axis=1, keepdims=True)
    probs_ref[0] = p


def _ffn_body(x_ref, idx_ref, pv_ref, w1_ref, w2_ref, out_ref):
    e = pl.program_id(1)

    @pl.when(e == 0)
    def _():
        out_ref[...] = jnp.zeros_like(out_ref)

    xb = x_ref[0]
    idxv = idx_ref[0, 0, 0]
    pv = pv_ref[0, 0, 0]
    K = idxv.shape[0]
    T = xb.shape[0]
    onehot = (lax.broadcasted_iota(jnp.int32, (K, T), 1) == idxv[:, None])
    onehot = onehot.astype(jnp.float32)
    xin = lax.dot_general(onehot, xb, (((1,), (0,)), ((), ())),
                          preferred_element_type=jnp.float32)
    h = lax.dot_general(xin, w1_ref[0], (((1,), (0,)), ((), ())),
                        preferred_element_type=jnp.float32)
    h = 0.5 * h * (1.0 + lax.erf(h * (1.0 / math.sqrt(2.0))))
    y = lax.dot_general(h, w2_ref[0], (((1,), (0,)), ((), ())),
                        preferred_element_type=jnp.float32)
    y = y * pv[:, None]
    out_ref[0] += lax.dot_general(onehot, y, (((0,), (0,)), ((), ())),
                                  preferred_element_type=jnp.float32)


def kernel(x, gate_w, w1, w2):
    B, T, C = x.shape
    E = gate_w.shape[0]
    H = w1.shape[2]
    K = int(2.0 * T / E)

    probs = pl.pallas_call(
        _gate_body,
        grid=(B,),
        in_specs=[
            pl.BlockSpec((1, T, C), lambda b: (b, 0, 0)),
            pl.BlockSpec((E, C), lambda b: (0, 0)),
        ],
        out_specs=pl.BlockSpec((1, T, E), lambda b: (b, 0, 0)),
        out_shape=jax.ShapeDtypeStruct((B, T, E), jnp.float32),
    )(x, gate_w)

    probs_t = jnp.transpose(probs, (0, 2, 1))
    pvals, idx = lax.top_k(probs_t, K)
    idx4 = idx.reshape(B, E, 1, K)
    pv4 = pvals.reshape(B, E, 1, K)

    out = pl.pallas_call(
        _ffn_body,
        grid=(B, E),
        in_specs=[
            pl.BlockSpec((1, T, C), lambda b, e: (b, 0, 0)),
            pl.BlockSpec((1, 1, 1, K), lambda b, e: (b, e, 0, 0)),
            pl.BlockSpec((1, 1, 1, K), lambda b, e: (b, e, 0, 0)),
            pl.BlockSpec((1, C, H), lambda b, e: (e, 0, 0)),
            pl.BlockSpec((1, H, C), lambda b, e: (e, 0, 0)),
        ],
        out_specs=pl.BlockSpec((1, T, C), lambda b, e: (b, 0, 0)),
        out_shape=jax.ShapeDtypeStruct((B, T, C), jnp.float32),
    )(x, idx4, pv4, w1, w2)
    return out
